# baseline (speedup 1.0000x reference)
import functools
import jax
import jax.numpy as jnp
from jax import lax
from jax.experimental import pallas as pl
from jax.experimental.pallas import tpu as pltpu
from jax.experimental.pallas import tpu_sc as plsc


def _sc_build_tile(col_w, row_w, h, w, dc, dr, nc, ns):
    d = dc + dr
    mesh = plsc.VectorSubcoreMesh(
        core_axis_name="c", subcore_axis_name="s", num_cores=nc, num_subcores=ns
    )

    @functools.partial(
        pl.kernel,
        mesh=mesh,
        out_type=jax.ShapeDtypeStruct((h, w, d), jnp.float32),
        scratch_types=[
            pltpu.VMEM((w, dc), jnp.float32),
            pltpu.VMEM((1, dr), jnp.float32),
            pltpu.VMEM((w, d), jnp.float32),
            pltpu.SemaphoreType.DMA,
        ],
    )
    def sc_k(col_hbm, row_hbm, tile_hbm, col_v, row_v, plane_v, sem):
        wid = lax.axis_index("s") * nc + lax.axis_index("c")
        pltpu.sync_copy(col_hbm, col_v)
        pltpu.sync_copy(row_hbm.at[pl.ds(wid, 1)], row_v)

        def build_row(iw, carry):
            for j in range(dc // 16):
                plane_v[iw, pl.ds(j * 16, 16)] = col_v[iw, pl.ds(j * 16, 16)]
            for j in range(dr // 16):
                plane_v[iw, pl.ds(dc + j * 16, 16)] = row_v[0, pl.ds(j * 16, 16)]
            return carry

        lax.fori_loop(0, w, build_row, 0)
        pltpu.async_copy(plane_v, tile_hbm.at[wid], sem).wait()

    return sc_k(col_w, row_w)


def kernel(pixel_values, row_weight, col_weight):
    if pixel_values.ndim != 4:
        raise ValueError('pixel_values must be a 4D tensor')
    b, h, w, _ = pixel_values.shape
    dr = row_weight.shape[1]
    dc = col_weight.shape[1]
    d = dc + dr

    row_w = row_weight[:h]
    col_w = col_weight[:w]

    nc, ns = 2, 16
    assert h == nc * ns

    tile = _sc_build_tile(col_w, row_w, h, w, dc, dr, nc, ns)

    def tc_body(tile_ref, out_hbm, sem):
        copies = [
            pltpu.make_async_copy(tile_ref, out_hbm.at[ib], sem) for ib in range(b)
        ]
        for c in copies:
            c.start()
        for c in copies:
            c.wait()

    out = pl.pallas_call(
        tc_body,
        in_specs=[pl.BlockSpec(memory_space=pltpu.VMEM)],
        out_specs=pl.BlockSpec(memory_space=pl.ANY),
        out_shape=jax.ShapeDtypeStruct((b, h, w, d), jnp.float32),
        scratch_shapes=[pltpu.SemaphoreType.DMA],
    )(tile)
    return out


# speedup vs baseline: 2.3884x; 2.3884x over previous
import jax
import jax.numpy as jnp
from jax.experimental import pallas as pl
from jax.experimental.pallas import tpu as pltpu


def kernel(pixel_values, row_weight, col_weight):
    if pixel_values.ndim != 4:
        raise ValueError('pixel_values must be a 4D tensor')
    b, h, w, _ = pixel_values.shape
    dr = row_weight.shape[1]
    dc = col_weight.shape[1]
    d = dc + dr

    row_w = row_weight[:h]
    col_w = col_weight[:w]

    nchunks = 2 if h % 2 == 0 else 1
    hc = h // nchunks

    def body(col_ref, row_ref, out_hbm, tile, sem):
        cw = col_ref[...]
        copies = []
        for k in range(nchunks):
            rw = row_ref[pl.ds(k * hc, hc), :]
            tile[pl.ds(k * hc, hc), :, :dc] = jnp.broadcast_to(
                cw[None, :, :], (hc, w, dc))
            tile[pl.ds(k * hc, hc), :, dc:] = jnp.broadcast_to(
                rw[:, None, :], (hc, w, dr))
            for ib in range(b):
                c = pltpu.make_async_copy(
                    tile.at[pl.ds(k * hc, hc)],
                    out_hbm.at[ib, pl.ds(k * hc, hc)],
                    sem,
                )
                c.start()
                copies.append(c)
        for c in copies:
            c.wait()

    out = pl.pallas_call(
        body,
        in_specs=[
            pl.BlockSpec(memory_space=pltpu.VMEM),
            pl.BlockSpec(memory_space=pltpu.VMEM),
        ],
        out_specs=pl.BlockSpec(memory_space=pl.ANY),
        out_shape=jax.ShapeDtypeStruct((b, h, w, d), jnp.float32),
        scratch_shapes=[
            pltpu.VMEM((h, w, d), jnp.float32),
            pltpu.SemaphoreType.DMA,
        ],
    )(col_w, row_w)
    return out


# speedup vs baseline: 2.4296x; 1.0173x over previous
import jax
import jax.numpy as jnp
from jax.experimental import pallas as pl
from jax.experimental.pallas import tpu as pltpu


def kernel(pixel_values, row_weight, col_weight):
    if pixel_values.ndim != 4:
        raise ValueError('pixel_values must be a 4D tensor')
    b, h, w, _ = pixel_values.shape
    dr = row_weight.shape[1]
    dc = col_weight.shape[1]
    d = dc + dr

    row_w = row_weight[:h]
    col_w = col_weight[:w]

    nchunks = 4 if h % 4 == 0 else 1
    hc = h // nchunks

    def body(col_ref, row_ref, out_hbm, tile, sem):
        cw = col_ref[...]
        copies = []
        for k in range(nchunks):
            rw = row_ref[pl.ds(k * hc, hc), :]
            tile[pl.ds(k * hc, hc), :, :dc] = jnp.broadcast_to(
                cw[None, :, :], (hc, w, dc))
            tile[pl.ds(k * hc, hc), :, dc:] = jnp.broadcast_to(
                rw[:, None, :], (hc, w, dr))
            for ib in range(b):
                c = pltpu.make_async_copy(
                    tile.at[pl.ds(k * hc, hc)],
                    out_hbm.at[ib, pl.ds(k * hc, hc)],
                    sem,
                )
                c.start()
                copies.append(c)
        for c in copies:
            c.wait()

    out = pl.pallas_call(
        body,
        in_specs=[
            pl.BlockSpec(memory_space=pltpu.VMEM),
            pl.BlockSpec(memory_space=pltpu.VMEM),
        ],
        out_specs=pl.BlockSpec(memory_space=pl.ANY),
        out_shape=jax.ShapeDtypeStruct((b, h, w, d), jnp.float32),
        scratch_shapes=[
            pltpu.VMEM((h, w, d), jnp.float32),
            pltpu.SemaphoreType.DMA,
        ],
    )(col_w, row_w)
    return out
